# baseline (speedup 1.0000x reference)
import functools
import jax
import jax.numpy as jnp
from jax import lax
from jax.experimental import pallas as pl
from jax.experimental.pallas import tpu as pltpu
from jax.experimental.pallas import tpu_sc as plsc

_N_E = 512
_C = 32


def _idx_body(z_ref, emb_ref, idx_ref):
    th = z_ref.shape[2]
    ts = th * z_ref.shape[3]
    zb = z_ref[0].reshape(_C, ts)
    emb = emb_ref[...]
    e_sq = jnp.sum(emb * emb, axis=1)
    z_sq = jnp.sum(zb * zb, axis=0)
    ez2 = jax.lax.dot_general(emb + emb, zb, (((1,), (0,)), ((), ())),
                              preferred_element_type=jnp.float32)
    dist = (z_sq[None, :] + e_sq[:, None]) - ez2
    m = jnp.min(dist, axis=0)
    eqf = (dist == m[None, :]).astype(jnp.float32)
    r8 = jax.lax.broadcasted_iota(jnp.int32, (8, _N_E), 0)
    c8 = jax.lax.broadcasted_iota(jnp.int32, (8, _N_E), 1)
    hi, lo = c8 >> 4, c8 & 15
    hi2, hl = hi * hi, hi * lo
    rows = jnp.where(r8 == 1, hi,
           jnp.where(r8 == 2, lo,
           jnp.where(r8 == 3, hi2 >> 2,
           jnp.where(r8 == 4, hi2 & 3,
           jnp.where(r8 == 5, hl >> 1,
           jnp.where(r8 == 6, hl & 1,
           jnp.where(r8 == 7, lo * lo, 1)))))))
    stats = jax.lax.dot_general(rows.astype(jnp.float32), eqf,
                                (((1,), (0,)), ((), ())),
                                preferred_element_type=jnp.float32)
    cnt = stats[0]
    si = 16.0 * stats[1] + stats[2]
    qi = (256.0 * (4.0 * stats[3] + stats[4])
          + 32.0 * (2.0 * stats[5] + stats[6]) + stats[7])
    delta = jnp.sqrt(jnp.maximum(qi + qi - si * si, 0.0))
    idx = jnp.where(cnt == 1.0, si, (si - delta) * 0.5).astype(jnp.int32)
    idx_ref[...] = idx.reshape(ts // 128, 128)


def _tc_indices(z, emb_weight):
    bs, c, h, w = z.shape
    th = 64
    ts = th * w
    return pl.pallas_call(
        _idx_body,
        grid=(bs,),
        in_specs=[pl.BlockSpec((1, c, th, w), lambda i: (i, 0, 0, 0)),
                  pl.BlockSpec((_N_E, _C), lambda i: (0, 0))],
        out_specs=pl.BlockSpec((ts // 128, 128), lambda i: (i, 0)),
        out_shape=jax.ShapeDtypeStruct((bs * h * w // 128, 128), jnp.int32),
    )(z, emb_weight)


def _make_sc_gather(bs, h, w):
    tok_per_worker = bs * h * w // 32
    groups = tok_per_worker // 16
    mesh = plsc.VectorSubcoreMesh(core_axis_name="c", subcore_axis_name="s")

    @functools.partial(
        pl.kernel, mesh=mesh,
        out_type=jax.ShapeDtypeStruct((bs, _C, h, 128), jnp.float32),
        scratch_types=[
            pltpu.VMEM((tok_per_worker,), jnp.int32),
            pltpu.VMEM((_C * _N_E,), jnp.float32),
            pltpu.VMEM((16, h // 2, 128), jnp.float32),
        ],
        compiler_params=pltpu.CompilerParams(use_tc_tiling_on_sc=False, needs_layout_passes=False),
    )
    def sc_gather(idx_hbm, embf_hbm, out_hbm, idx_v, emb_v, out_v):
        wid = lax.axis_index("s") * 2 + lax.axis_index("c")
        b = wid // 2
        half = wid % 2
        pltpu.sync_copy(idx_hbm.at[pl.ds(wid * tok_per_worker, tok_per_worker)],
                        idx_v)
        pltpu.sync_copy(embf_hbm, emb_v)
        for cg in range(2):
            def g_body(g, carry):
                idx16 = idx_v[pl.ds(g * 16, 16)]
                h_rel = g // 4
                lc = (g % 4) * 16
                for c_rel in range(16):
                    c = cg * 16 + c_rel
                    vals = plsc.load_gather(emb_v, [idx16 + c * _N_E])
                    out_v[c_rel, h_rel, pl.ds(lc, 16)] = vals
                return carry
            lax.fori_loop(0, groups, g_body, 0)
            for c_rel in range(16):
                c = cg * 16 + c_rel
                pltpu.sync_copy(out_v.at[c_rel],
                                out_hbm.at[b, c, pl.ds(half * (h // 2), h // 2), :])

    return sc_gather


def kernel(z, emb_weight):
    bs, c, h, w = z.shape
    idx = _tc_indices(z, emb_weight)
    idx_flat = idx.reshape(-1)
    emb_flat = emb_weight.T.reshape(-1)
    out_wide = _make_sc_gather(bs, h, w)(idx_flat, emb_flat)
    return out_wide[..., :w]
